# baseline (speedup 1.0000x reference)
import jax
import jax.numpy as jnp
from jax import lax
from jax.experimental import pallas as pl

N = 20000
NPAD = 20480
K = 16
BR = 512
BC = 2048
DPAD = 128


def _dist_body(a_ref, b_ref, o_ref):
    j = pl.program_id(1)
    a_blk = a_ref[...]
    b_blk = b_ref[...]
    ab = lax.dot_general(
        a_blk, b_blk,
        dimension_numbers=(((1,), (1,)), ((), ())),
        preferred_element_type=jnp.float32,
    )
    a2 = jnp.sum(a_blk * a_blk, axis=1, keepdims=True)
    b2 = jnp.sum(b_blk * b_blk, axis=1)[None, :]
    sq = (a2 + b2) - 2.0 * ab
    d = jnp.sqrt(jnp.maximum(sq, 0.0))
    col = j * BC + lax.broadcasted_iota(jnp.int32, (BR, BC), 1)
    d = jnp.where(col >= N, jnp.inf, d)
    o_ref[...] = d


def _distances(a_pad):
    return pl.pallas_call(
        _dist_body,
        grid=(NPAD // BR, NPAD // BC),
        in_specs=[
            pl.BlockSpec((BR, DPAD), lambda i, j: (i, 0)),
            pl.BlockSpec((BC, DPAD), lambda i, j: (j, 0)),
        ],
        out_specs=pl.BlockSpec((BR, BC), lambda i, j: (i, j)),
        out_shape=jax.ShapeDtypeStruct((NPAD, NPAD), jnp.float32),
    )(a_pad, a_pad)


def kernel(barycenters):
    a_pad = jnp.zeros((NPAD, DPAD), jnp.float32)
    a_pad = a_pad.at[:N, :3].set(barycenters)
    d = _distances(a_pad)
    _, idx = lax.top_k(-d[:N], K)
    return idx.astype(jnp.float32)


# speedup vs baseline: 1.0095x; 1.0095x over previous
import functools

import jax
import jax.numpy as jnp
from jax import lax
from jax.experimental import pallas as pl
from jax.experimental.pallas import tpu as pltpu
from jax.experimental.pallas import tpu_sc as plsc

N = 20000
NPAD = 20480
K = 16
BR = 512
BC = 2048

NW = 32
RPW = N // NW
VPR = N // 16


def _dist_body(a_ref, bt_ref, o_ref):
    j = pl.program_id(1)
    a_blk = a_ref[...]
    bt_blk = bt_ref[...]
    ab = lax.dot_general(
        a_blk, bt_blk,
        dimension_numbers=(((1,), (0,)), ((), ())),
        preferred_element_type=jnp.float32,
    )
    a2 = jnp.sum(a_blk * a_blk, axis=1, keepdims=True)
    b2 = jnp.sum(bt_blk * bt_blk, axis=0)[None, :]
    sq = (a2 + b2) - 2.0 * ab
    d = jnp.sqrt(jnp.maximum(sq, 0.0))
    col = j * BC + lax.broadcasted_iota(jnp.int32, (BR, BC), 1)
    d = jnp.where(col >= N, jnp.inf, d)
    o_ref[...] = d


def _distances(a_pad, bt_pad):
    return pl.pallas_call(
        _dist_body,
        grid=(NPAD // BR, NPAD // BC),
        in_specs=[
            pl.BlockSpec((BR, 3), lambda i, j: (i, 0)),
            pl.BlockSpec((3, BC), lambda i, j: (0, j)),
        ],
        out_specs=pl.BlockSpec((BR, BC), lambda i, j: (i, j)),
        out_shape=jax.ShapeDtypeStruct((NPAD, NPAD), jnp.float32),
    )(a_pad, bt_pad)


def _lex_lt(ka, ia, kb, ib):
    return (ka < kb) | ((ka == kb) & (ia < ib))


def _row_topk(buf, nrow_base):
    lane = lax.iota(jnp.int32, 16)

    lane15 = jnp.full((16,), 15, jnp.int32)

    def body(vi, carry):
        kv, ki, tv = carry
        v = buf[pl.ds(vi * 16, 16)]
        cnt = plsc.all_reduce_population_count(v <= tv)
        hit = cnt[0] > 0

        def merge(kv, ki, tv):
            nidx = vi * 16 + lane
            nk, ni = plsc.sort_key_val(v, nidx, descending=True)
            lt = _lex_lt(nk, ni, kv, ki)
            lk = jnp.where(lt, nk, kv)
            li = jnp.where(lt, ni, ki)
            kv2, ki2 = plsc.sort_key_val(lk, li)
            tv2 = lax.gather(
                kv2, lane15[:, None],
                dimension_numbers=lax.GatherDimensionNumbers(
                    offset_dims=(), collapsed_slice_dims=(0,),
                    start_index_map=(0,)),
                slice_sizes=(1,),
                mode=lax.GatherScatterMode.PROMISE_IN_BOUNDS)
            return kv2, ki2, tv2

        def skip(kv, ki, tv):
            return kv, ki, tv

        return lax.cond(hit, merge, skip, kv, ki, tv)

    kv0 = jnp.full((16,), jnp.inf, jnp.float32)
    ki0 = jnp.zeros((16,), jnp.int32)
    tv0 = jnp.full((16,), jnp.inf, jnp.float32)
    kv, ki, tv = lax.fori_loop(0, VPR, body, (kv0, ki0, tv0))
    del nrow_base
    return kv, ki


def _topk_sc(d):
    mesh = plsc.VectorSubcoreMesh(core_axis_name="c", subcore_axis_name="s")

    @functools.partial(
        pl.kernel,
        mesh=mesh,
        compiler_params=pltpu.CompilerParams(needs_layout_passes=False),
        out_type=jax.ShapeDtypeStruct((N * K,), jnp.float32),
        scratch_types=[
            pltpu.VMEM((NPAD,), jnp.float32),
            pltpu.VMEM((NPAD,), jnp.float32),
            pltpu.VMEM((RPW * K,), jnp.float32),
            pltpu.SemaphoreType.DMA,
            pltpu.SemaphoreType.DMA,
        ],
    )
    def k(d_hbm, out_hbm, buf0, buf1, out_v, sem0, sem1):
        wid = lax.axis_index("s") * 2 + lax.axis_index("c")
        row0 = wid * RPW
        bufs = (buf0, buf1)
        sems = (sem0, sem1)

        pltpu.async_copy(d_hbm.at[row0], buf0, sem0)

        def outer(i2, _):
            for b in range(2):
                r = i2 * 2 + b
                row = row0 + r
                pltpu.async_copy(d_hbm.at[row + 1], bufs[1 - b], sems[1 - b])
                pltpu.make_async_copy(d_hbm.at[row], bufs[b], sems[b]).wait()
                kv, ki = _row_topk(bufs[b], row)

                @pl.when(r < RPW)
                def _():
                    out_v[pl.ds(r * K, K)] = ki.astype(jnp.float32)

            return 0

        lax.fori_loop(0, (RPW + 2) // 2, outer, 0)
        pltpu.make_async_copy(d_hbm.at[row0 + RPW + 1], buf0, sem0).wait()
        pltpu.sync_copy(out_v, out_hbm.at[pl.ds(row0 * K, RPW * K)])

    out = k(d)
    return out.reshape(N, K)


def kernel(barycenters):
    a_pad = jnp.zeros((NPAD, 3), jnp.float32)
    a_pad = a_pad.at[:N].set(barycenters)
    bt_pad = jnp.zeros((3, NPAD), jnp.float32)
    bt_pad = bt_pad.at[:, :N].set(barycenters.T)
    d = _distances(a_pad, bt_pad)
    return _topk_sc(d)


# speedup vs baseline: 3.1432x; 3.1137x over previous
import functools

import jax
import jax.numpy as jnp
from jax import lax
from jax.experimental import pallas as pl
from jax.experimental.pallas import tpu as pltpu
from jax.experimental.pallas import tpu_sc as plsc

N = 20000
NPAD = 20480
K = 16
BR = 512
BC = 2048
NB = NPAD // 16

NW = 32
RPW = N // NW
MV = NB // 16


def _dist_body(a_ref, bt_ref, o_ref, m_ref):
    j = pl.program_id(1)
    a_blk = a_ref[...]
    bt_blk = bt_ref[...]
    ab = lax.dot_general(
        a_blk, bt_blk,
        dimension_numbers=(((1,), (0,)), ((), ())),
        preferred_element_type=jnp.float32,
    )
    a2 = jnp.sum(a_blk * a_blk, axis=1, keepdims=True)
    b2 = jnp.sum(bt_blk * bt_blk, axis=0)[None, :]
    sq = (a2 + b2) - 2.0 * ab
    d = jnp.sqrt(jnp.maximum(sq, 0.0))
    col = j * BC + lax.broadcasted_iota(jnp.int32, (BR, BC), 1)
    d = jnp.where(col >= N, jnp.inf, d)
    o_ref[...] = d
    m_ref[...] = jnp.min(d.reshape(BR, BC // 16, 16), axis=2)


def _distances(a_pad, bt_pad):
    return pl.pallas_call(
        _dist_body,
        grid=(NPAD // BR, NPAD // BC),
        in_specs=[
            pl.BlockSpec((BR, 3), lambda i, j: (i, 0)),
            pl.BlockSpec((3, BC), lambda i, j: (0, j)),
        ],
        out_specs=[
            pl.BlockSpec((BR, BC), lambda i, j: (i, j)),
            pl.BlockSpec((BR, BC // 16), lambda i, j: (i, j)),
        ],
        out_shape=[
            jax.ShapeDtypeStruct((NPAD, NPAD), jnp.float32),
            jax.ShapeDtypeStruct((NPAD, NB), jnp.float32),
        ],
    )(a_pad, bt_pad)


def _lex_lt(ka, ia, kb, ib):
    return (ka < kb) | ((ka == kb) & (ia < ib))


def _row_topk(dbuf, mbuf, bids):
    lane = lax.iota(jnp.int32, 16)

    def abody(i, em):
        return jnp.minimum(em, mbuf[pl.ds(i * 16, 16)])

    em = lax.fori_loop(1, MV, abody, mbuf[pl.ds(0, 16)], unroll=8)
    tvec = jnp.full((16,), jnp.max(em), jnp.float32)

    def bbody(mb, ptr):
        mv = mbuf[pl.ds(mb * 16, 16)]
        msk = mv <= tvec
        bidv = mb * 16 + lane
        plsc.store_compressed(bids.at[pl.ds(ptr, 16)], bidv, mask=msk)
        cnt = plsc.all_reduce_population_count(msk)
        return ptr + cnt[0]

    nh = lax.fori_loop(0, MV, bbody, 0, unroll=4)

    def cbody(hi, carry):
        kv, ki, tv = carry
        bid = bids[pl.ds(hi, 16)][0]
        v = dbuf[pl.ds(bid * 16, 16)]
        cnt = plsc.all_reduce_population_count(v <= tv)
        hit = cnt[0] > 0

        def merge(kv, ki, tv):
            nidx = bid * 16 + lane
            nk, ni = plsc.sort_key_val(v, nidx, descending=True)
            lt = _lex_lt(nk, ni, kv, ki)
            lk = jnp.where(lt, nk, kv)
            li = jnp.where(lt, ni, ki)
            kv2, ki2 = plsc.sort_key_val(lk, li)
            tv2 = lax.gather(
                kv2, jnp.full((16, 1), 15, jnp.int32),
                dimension_numbers=lax.GatherDimensionNumbers(
                    offset_dims=(), collapsed_slice_dims=(0,),
                    start_index_map=(0,)),
                slice_sizes=(1,),
                mode=lax.GatherScatterMode.PROMISE_IN_BOUNDS)
            return kv2, ki2, tv2

        def skip(kv, ki, tv):
            return kv, ki, tv

        return lax.cond(hit, merge, skip, kv, ki, tv)

    kv0 = jnp.full((16,), jnp.inf, jnp.float32)
    ki0 = jnp.zeros((16,), jnp.int32)
    kv, ki, _ = lax.fori_loop(0, nh, cbody, (kv0, ki0, tvec))
    return kv, ki


def _topk_sc(d, m):
    mesh = plsc.VectorSubcoreMesh(core_axis_name="c", subcore_axis_name="s")

    @functools.partial(
        pl.kernel,
        mesh=mesh,
        compiler_params=pltpu.CompilerParams(needs_layout_passes=False),
        out_type=jax.ShapeDtypeStruct((N * K,), jnp.float32),
        scratch_types=[
            pltpu.VMEM((NPAD,), jnp.float32),
            pltpu.VMEM((NPAD,), jnp.float32),
            pltpu.VMEM((NB,), jnp.float32),
            pltpu.VMEM((NB,), jnp.float32),
            pltpu.VMEM((NB + 16,), jnp.int32),
            pltpu.VMEM((RPW * K,), jnp.float32),
            pltpu.SemaphoreType.DMA,
            pltpu.SemaphoreType.DMA,
            pltpu.SemaphoreType.DMA,
            pltpu.SemaphoreType.DMA,
        ],
    )
    def k(d_hbm, m_hbm, out_hbm, dbuf0, dbuf1, mbuf0, mbuf1, bids, out_v,
          dsem0, dsem1, msem0, msem1):
        wid = lax.axis_index("s") * 2 + lax.axis_index("c")
        row0 = wid * RPW
        dbufs = (dbuf0, dbuf1)
        msems = (msem0, msem1)
        mbufs = (mbuf0, mbuf1)
        dsems = (dsem0, dsem1)

        pltpu.async_copy(d_hbm.at[row0], dbuf0, dsem0)
        pltpu.async_copy(m_hbm.at[row0], mbuf0, msem0)

        def outer(i2, _):
            for b in range(2):
                r = i2 * 2 + b
                row = row0 + r
                pltpu.async_copy(d_hbm.at[row + 1], dbufs[1 - b], dsems[1 - b])
                pltpu.async_copy(m_hbm.at[row + 1], mbufs[1 - b], msems[1 - b])
                pltpu.make_async_copy(d_hbm.at[row], dbufs[b], dsems[b]).wait()
                pltpu.make_async_copy(m_hbm.at[row], mbufs[b], msems[b]).wait()
                kv, ki = _row_topk(dbufs[b], mbufs[b], bids)

                @pl.when(r < RPW)
                def _():
                    out_v[pl.ds(r * K, K)] = ki.astype(jnp.float32)

            return 0

        lax.fori_loop(0, (RPW + 2) // 2, outer, 0)
        pltpu.make_async_copy(d_hbm.at[row0 + RPW + 1], dbuf0, dsem0).wait()
        pltpu.make_async_copy(m_hbm.at[row0 + RPW + 1], mbuf0, msem0).wait()
        pltpu.sync_copy(out_v, out_hbm.at[pl.ds(row0 * K, RPW * K)])

    out = k(d, m)
    return out.reshape(N, K)


def kernel(barycenters):
    a_pad = jnp.zeros((NPAD, 3), jnp.float32)
    a_pad = a_pad.at[:N].set(barycenters)
    bt_pad = jnp.zeros((3, NPAD), jnp.float32)
    bt_pad = bt_pad.at[:, :N].set(barycenters.T)
    d, m = _distances(a_pad, bt_pad)
    return _topk_sc(d, m)


# speedup vs baseline: 7.8022x; 2.4822x over previous
import functools

import jax
import jax.numpy as jnp
from jax import lax
from jax.experimental import pallas as pl
from jax.experimental.pallas import tpu as pltpu
from jax.experimental.pallas import tpu_sc as plsc

N = 20000
NPAD = 20480
K = 16
BR = 512
BC = 2048
NB = NPAD // 16
MARGIN = 1e-4

NW = 32
RPW = N // NW
MV = NB // 16


def _sq_dist(a_blk, bt_blk):
    ab = lax.dot_general(
        a_blk, bt_blk,
        dimension_numbers=(((1,), (0,)), ((), ())),
        preferred_element_type=jnp.float32,
    )
    a2 = jnp.sum(a_blk * a_blk, axis=1, keepdims=True)
    b2 = jnp.sum(bt_blk * bt_blk, axis=0)[None, :]
    sq = (a2 + b2) - 2.0 * ab
    return jnp.sqrt(jnp.maximum(sq, 0.0))


def _dist_body(a_ref, bt_ref, o_ref):
    j = pl.program_id(1)
    d = _sq_dist(a_ref[...], bt_ref[...])
    col = j * BC + lax.broadcasted_iota(jnp.int32, (BR, BC), 1)
    o_ref[...] = jnp.where(col >= N, jnp.inf, d)


def _distances(a_pad, bt_pad):
    return pl.pallas_call(
        _dist_body,
        grid=(NPAD // BR, NPAD // BC),
        in_specs=[
            pl.BlockSpec((BR, 3), lambda i, j: (i, 0)),
            pl.BlockSpec((3, BC), lambda i, j: (0, j)),
        ],
        out_specs=pl.BlockSpec((BR, BC), lambda i, j: (i, j)),
        out_shape=jax.ShapeDtypeStruct((NPAD, NPAD), jnp.float32),
    )(a_pad, bt_pad)


def _mblock_body(a_ref, bt_ref, m_ref):
    jc = pl.program_id(1)
    d = _sq_dist(a_ref[...], bt_ref[...])
    cc = jc * BC + lax.broadcasted_iota(jnp.int32, (BC, BR), 0)
    d = jnp.where(cc >= N, jnp.inf, d)
    gm = jnp.min(d.reshape(BC // 16, 16, BR), axis=1)
    m_ref[...] = gm.T


def _blockmins(a_pad, bt_pad):
    return pl.pallas_call(
        _mblock_body,
        grid=(NPAD // BR, NPAD // BC),
        in_specs=[
            pl.BlockSpec((BC, 3), lambda i, j: (j, 0)),
            pl.BlockSpec((3, BR), lambda i, j: (0, i)),
        ],
        out_specs=pl.BlockSpec((BR, BC // 16), lambda i, j: (i, j)),
        out_shape=jax.ShapeDtypeStruct((NPAD, NB), jnp.float32),
    )(a_pad, bt_pad)


def _lex_lt(ka, ia, kb, ib):
    return (ka < kb) | ((ka == kb) & (ia < ib))


def _row_topk(dbuf, mbuf, bids):
    lane = lax.iota(jnp.int32, 16)

    def abody(i, em):
        return jnp.minimum(em, mbuf[pl.ds(i * 16, 16)])

    em = lax.fori_loop(1, MV, abody, mbuf[pl.ds(0, 16)], unroll=8)
    tvec = jnp.full((16,), jnp.max(em) + MARGIN, jnp.float32)

    def bbody(mb, ptr):
        mv = mbuf[pl.ds(mb * 16, 16)]
        msk = mv <= tvec
        bidv = mb * 16 + lane
        plsc.store_compressed(bids.at[pl.ds(ptr, 16)], bidv, mask=msk)
        cnt = plsc.all_reduce_population_count(msk)
        return ptr + cnt[0]

    nh = lax.fori_loop(0, MV, bbody, 0, unroll=4)

    def cbody(hi, carry):
        kv, ki, tv = carry
        bid = bids[pl.ds(hi, 16)][0]
        v = dbuf[pl.ds(bid * 16, 16)]
        cnt = plsc.all_reduce_population_count(v <= tv)
        hit = cnt[0] > 0

        def merge(kv, ki, tv):
            nidx = bid * 16 + lane
            nk, ni = plsc.sort_key_val(v, nidx, descending=True)
            lt = _lex_lt(nk, ni, kv, ki)
            lk = jnp.where(lt, nk, kv)
            li = jnp.where(lt, ni, ki)
            kv2, ki2 = plsc.sort_key_val(lk, li)
            tv2 = lax.gather(
                kv2, jnp.full((16, 1), 15, jnp.int32),
                dimension_numbers=lax.GatherDimensionNumbers(
                    offset_dims=(), collapsed_slice_dims=(0,),
                    start_index_map=(0,)),
                slice_sizes=(1,),
                mode=lax.GatherScatterMode.PROMISE_IN_BOUNDS)
            return kv2, ki2, tv2

        def skip(kv, ki, tv):
            return kv, ki, tv

        return lax.cond(hit, merge, skip, kv, ki, tv)

    kv0 = jnp.full((16,), jnp.inf, jnp.float32)
    ki0 = jnp.zeros((16,), jnp.int32)
    kv, ki, _ = lax.fori_loop(0, nh, cbody, (kv0, ki0, tvec))
    return kv, ki


def _topk_sc(d, m):
    mesh = plsc.VectorSubcoreMesh(core_axis_name="c", subcore_axis_name="s")

    @functools.partial(
        pl.kernel,
        mesh=mesh,
        compiler_params=pltpu.CompilerParams(needs_layout_passes=False),
        out_type=jax.ShapeDtypeStruct((N * K,), jnp.float32),
        scratch_types=[
            pltpu.VMEM((NPAD,), jnp.float32),
            pltpu.VMEM((NPAD,), jnp.float32),
            pltpu.VMEM((NB,), jnp.float32),
            pltpu.VMEM((NB,), jnp.float32),
            pltpu.VMEM((NB + 16,), jnp.int32),
            pltpu.VMEM((RPW * K,), jnp.float32),
            pltpu.SemaphoreType.DMA,
            pltpu.SemaphoreType.DMA,
            pltpu.SemaphoreType.DMA,
            pltpu.SemaphoreType.DMA,
        ],
    )
    def k(d_hbm, m_hbm, out_hbm, dbuf0, dbuf1, mbuf0, mbuf1, bids, out_v,
          dsem0, dsem1, msem0, msem1):
        wid = lax.axis_index("s") * 2 + lax.axis_index("c")
        row0 = wid * RPW
        dbufs = (dbuf0, dbuf1)
        msems = (msem0, msem1)
        mbufs = (mbuf0, mbuf1)
        dsems = (dsem0, dsem1)

        pltpu.async_copy(d_hbm.at[row0], dbuf0, dsem0)
        pltpu.async_copy(m_hbm.at[row0], mbuf0, msem0)

        def outer(i2, _):
            for b in range(2):
                r = i2 * 2 + b
                row = row0 + r
                pltpu.async_copy(d_hbm.at[row + 1], dbufs[1 - b], dsems[1 - b])
                pltpu.async_copy(m_hbm.at[row + 1], mbufs[1 - b], msems[1 - b])
                pltpu.make_async_copy(d_hbm.at[row], dbufs[b], dsems[b]).wait()
                pltpu.make_async_copy(m_hbm.at[row], mbufs[b], msems[b]).wait()
                kv, ki = _row_topk(dbufs[b], mbufs[b], bids)

                @pl.when(r < RPW)
                def _():
                    out_v[pl.ds(r * K, K)] = ki.astype(jnp.float32)

            return 0

        lax.fori_loop(0, (RPW + 2) // 2, outer, 0)
        pltpu.make_async_copy(d_hbm.at[row0 + RPW + 1], dbuf0, dsem0).wait()
        pltpu.make_async_copy(m_hbm.at[row0 + RPW + 1], mbuf0, msem0).wait()
        pltpu.sync_copy(out_v, out_hbm.at[pl.ds(row0 * K, RPW * K)])

    out = k(d, m)
    return out.reshape(N, K)


def kernel(barycenters):
    a_pad = jnp.zeros((NPAD, 3), jnp.float32)
    a_pad = a_pad.at[:N].set(barycenters)
    bt_pad = jnp.zeros((3, NPAD), jnp.float32)
    bt_pad = bt_pad.at[:, :N].set(barycenters.T)
    d = _distances(a_pad, bt_pad)
    m = _blockmins(a_pad, bt_pad)
    return _topk_sc(d, m)


# speedup vs baseline: 8.4724x; 1.0859x over previous
import functools

import jax
import jax.numpy as jnp
from jax import lax
from jax.experimental import pallas as pl
from jax.experimental.pallas import tpu as pltpu
from jax.experimental.pallas import tpu_sc as plsc

N = 20000
NPAD = 20480
K = 16
BR = 512
BC = 2048
NB = NPAD // 16
MARGIN = 1e-4

NW = 32
RPW = N // NW
MV = NB // 16


def _sq_dist(a_blk, bt_blk):
    ab = lax.dot_general(
        a_blk, bt_blk,
        dimension_numbers=(((1,), (0,)), ((), ())),
        preferred_element_type=jnp.float32,
    )
    a2 = jnp.sum(a_blk * a_blk, axis=1, keepdims=True)
    b2 = jnp.sum(bt_blk * bt_blk, axis=0)[None, :]
    return (a2 + b2) - 2.0 * ab


def _dist_body(a_ref, bt_ref, o_ref):
    j = pl.program_id(1)
    d = jnp.sqrt(jnp.maximum(_sq_dist(a_ref[...], bt_ref[...]), 0.0))
    col = j * BC + lax.broadcasted_iota(jnp.int32, (BR, BC), 1)
    o_ref[...] = jnp.where(col >= N, jnp.inf, d)


def _distances(a_pad, bt_pad):
    return pl.pallas_call(
        _dist_body,
        grid=(NPAD // BR, NPAD // BC),
        in_specs=[
            pl.BlockSpec((BR, 3), lambda i, j: (i, 0)),
            pl.BlockSpec((3, BC), lambda i, j: (0, j)),
        ],
        out_specs=pl.BlockSpec((BR, BC), lambda i, j: (i, j)),
        out_shape=jax.ShapeDtypeStruct((NPAD, NPAD), jnp.float32),
    )(a_pad, bt_pad)


def _mblock_body(a_ref, bt_ref, m_ref):
    jc = pl.program_id(1)
    sq = _sq_dist(a_ref[...], bt_ref[...])
    cc = jc * BC + lax.broadcasted_iota(jnp.int32, (BC, BR), 0)
    sq = jnp.where(cc >= N, jnp.inf, sq)
    gm = jnp.min(sq.reshape(BC // 16, 16, BR), axis=1)
    m_ref[...] = jnp.sqrt(jnp.maximum(gm, 0.0)).T


def _blockmins(a_pad, bt_pad):
    return pl.pallas_call(
        _mblock_body,
        grid=(NPAD // BR, NPAD // BC),
        in_specs=[
            pl.BlockSpec((BC, 3), lambda i, j: (j, 0)),
            pl.BlockSpec((3, BR), lambda i, j: (0, i)),
        ],
        out_specs=pl.BlockSpec((BR, BC // 16), lambda i, j: (i, j)),
        out_shape=jax.ShapeDtypeStruct((NPAD, NB), jnp.float32),
    )(a_pad, bt_pad)


def _lex_lt(ka, ia, kb, ib):
    return (ka < kb) | ((ka == kb) & (ia < ib))


def _row_topk(dbuf, mbuf, bids):
    lane = lax.iota(jnp.int32, 16)

    def abody(i, em):
        return jnp.minimum(em, mbuf[pl.ds(i * 16, 16)])

    em = lax.fori_loop(1, MV, abody, mbuf[pl.ds(0, 16)], unroll=8)
    tvec = jnp.full((16,), jnp.max(em) + MARGIN, jnp.float32)

    def bbody(mb, ptr):
        mv = mbuf[pl.ds(mb * 16, 16)]
        msk = mv <= tvec
        bidv = mb * 16 + lane
        plsc.store_compressed(bids.at[pl.ds(ptr, 16)], bidv, mask=msk)
        cnt = plsc.all_reduce_population_count(msk)
        return ptr + cnt[0]

    nh = lax.fori_loop(0, MV, bbody, 0, unroll=4)

    def cbody(hi, carry):
        kv, ki, tv = carry
        bid = bids[pl.ds(hi, 16)][0]
        v = dbuf[pl.ds(bid * 16, 16)]
        cnt = plsc.all_reduce_population_count(v <= tv)
        hit = cnt[0] > 0

        def merge(kv, ki, tv):
            nidx = bid * 16 + lane
            nk, ni = plsc.sort_key_val(v, nidx, descending=True)
            lt = _lex_lt(nk, ni, kv, ki)
            lk = jnp.where(lt, nk, kv)
            li = jnp.where(lt, ni, ki)
            kv2, ki2 = plsc.sort_key_val(lk, li)
            tv2 = lax.gather(
                kv2, jnp.full((16, 1), 15, jnp.int32),
                dimension_numbers=lax.GatherDimensionNumbers(
                    offset_dims=(), collapsed_slice_dims=(0,),
                    start_index_map=(0,)),
                slice_sizes=(1,),
                mode=lax.GatherScatterMode.PROMISE_IN_BOUNDS)
            return kv2, ki2, tv2

        def skip(kv, ki, tv):
            return kv, ki, tv

        return lax.cond(hit, merge, skip, kv, ki, tv)

    kv0 = jnp.full((16,), jnp.inf, jnp.float32)
    ki0 = jnp.zeros((16,), jnp.int32)
    kv, ki, _ = lax.fori_loop(0, nh, cbody, (kv0, ki0, tvec))
    return kv, ki


def _topk_sc(d, m):
    mesh = plsc.VectorSubcoreMesh(core_axis_name="c", subcore_axis_name="s")

    @functools.partial(
        pl.kernel,
        mesh=mesh,
        compiler_params=pltpu.CompilerParams(needs_layout_passes=False),
        out_type=jax.ShapeDtypeStruct((N * K,), jnp.float32),
        scratch_types=[
            pltpu.VMEM((NPAD,), jnp.float32),
            pltpu.VMEM((NPAD,), jnp.float32),
            pltpu.VMEM((NB,), jnp.float32),
            pltpu.VMEM((NB,), jnp.float32),
            pltpu.VMEM((NB + 16,), jnp.int32),
            pltpu.VMEM((RPW * K,), jnp.float32),
            pltpu.SemaphoreType.DMA,
            pltpu.SemaphoreType.DMA,
            pltpu.SemaphoreType.DMA,
            pltpu.SemaphoreType.DMA,
        ],
    )
    def k(d_hbm, m_hbm, out_hbm, dbuf0, dbuf1, mbuf0, mbuf1, bids, out_v,
          dsem0, dsem1, msem0, msem1):
        wid = lax.axis_index("s") * 2 + lax.axis_index("c")
        row0 = wid * RPW
        dbufs = (dbuf0, dbuf1)
        msems = (msem0, msem1)
        mbufs = (mbuf0, mbuf1)
        dsems = (dsem0, dsem1)

        pltpu.async_copy(d_hbm.at[row0], dbuf0, dsem0)
        pltpu.async_copy(m_hbm.at[row0], mbuf0, msem0)

        def outer(i2, _):
            for b in range(2):
                r = i2 * 2 + b
                row = row0 + r
                pltpu.async_copy(d_hbm.at[row + 1], dbufs[1 - b], dsems[1 - b])
                pltpu.async_copy(m_hbm.at[row + 1], mbufs[1 - b], msems[1 - b])
                pltpu.make_async_copy(d_hbm.at[row], dbufs[b], dsems[b]).wait()
                pltpu.make_async_copy(m_hbm.at[row], mbufs[b], msems[b]).wait()
                kv, ki = _row_topk(dbufs[b], mbufs[b], bids)

                @pl.when(r < RPW)
                def _():
                    out_v[pl.ds(r * K, K)] = ki.astype(jnp.float32)

            return 0

        lax.fori_loop(0, (RPW + 2) // 2, outer, 0)
        pltpu.make_async_copy(d_hbm.at[row0 + RPW + 1], dbuf0, dsem0).wait()
        pltpu.make_async_copy(m_hbm.at[row0 + RPW + 1], mbuf0, msem0).wait()
        pltpu.sync_copy(out_v, out_hbm.at[pl.ds(row0 * K, RPW * K)])

    out = k(d, m)
    return out.reshape(N, K)


def kernel(barycenters):
    a_pad = jnp.zeros((NPAD, 3), jnp.float32)
    a_pad = a_pad.at[:N].set(barycenters)
    bt_pad = jnp.zeros((3, NPAD), jnp.float32)
    bt_pad = bt_pad.at[:, :N].set(barycenters.T)
    d = _distances(a_pad, bt_pad)
    m = _blockmins(a_pad, bt_pad)
    return _topk_sc(d, m)


# speedup vs baseline: 9.8216x; 1.1592x over previous
import functools

import jax
import jax.numpy as jnp
from jax import lax
from jax.experimental import pallas as pl
from jax.experimental.pallas import tpu as pltpu
from jax.experimental.pallas import tpu_sc as plsc

N = 20000
NPAD = 20480
K = 16
BR = 512
BC = 2048
NB = NPAD // 16
MARGIN = 1e-4

CCAP = 1024
NW = 32
RPW = N // NW
MV = NB // 16


def _sq_dist(a_blk, bt_blk):
    ab = lax.dot_general(
        a_blk, bt_blk,
        dimension_numbers=(((1,), (0,)), ((), ())),
        preferred_element_type=jnp.float32,
    )
    a2 = jnp.sum(a_blk * a_blk, axis=1, keepdims=True)
    b2 = jnp.sum(bt_blk * bt_blk, axis=0)[None, :]
    return (a2 + b2) - 2.0 * ab


def _dist_body(a_ref, bt_ref, o_ref):
    j = pl.program_id(1)
    d = jnp.sqrt(jnp.maximum(_sq_dist(a_ref[...], bt_ref[...]), 0.0))
    col = j * BC + lax.broadcasted_iota(jnp.int32, (BR, BC), 1)
    o_ref[...] = jnp.where(col >= N, jnp.inf, d)


def _distances(a_pad, bt_pad):
    return pl.pallas_call(
        _dist_body,
        grid=(NPAD // BR, NPAD // BC),
        in_specs=[
            pl.BlockSpec((BR, 3), lambda i, j: (i, 0)),
            pl.BlockSpec((3, BC), lambda i, j: (0, j)),
        ],
        out_specs=pl.BlockSpec((BR, BC), lambda i, j: (i, j)),
        out_shape=jax.ShapeDtypeStruct((NPAD, NPAD), jnp.float32),
    )(a_pad, bt_pad)


def _mblock_body(a_ref, bt_ref, m_ref):
    jc = pl.program_id(1)
    sq = _sq_dist(a_ref[...], bt_ref[...])
    cc = jc * BC + lax.broadcasted_iota(jnp.int32, (BC, BR), 0)
    sq = jnp.where(cc >= N, jnp.inf, sq)
    gm = jnp.min(sq.reshape(BC // 16, 16, BR), axis=1)
    m_ref[...] = jnp.sqrt(jnp.maximum(gm, 0.0)).T


def _blockmins(a_pad, bt_pad):
    return pl.pallas_call(
        _mblock_body,
        grid=(NPAD // BR, NPAD // BC),
        in_specs=[
            pl.BlockSpec((BC, 3), lambda i, j: (j, 0)),
            pl.BlockSpec((3, BR), lambda i, j: (0, i)),
        ],
        out_specs=pl.BlockSpec((BR, BC // 16), lambda i, j: (i, j)),
        out_shape=jax.ShapeDtypeStruct((NPAD, NB), jnp.float32),
    )(a_pad, bt_pad)


def _lex_lt(ka, ia, kb, ib):
    return (ka < kb) | ((ka == kb) & (ia < ib))


def _row_topk(dbuf, mbuf, bids, cvals, cidx):
    lane = lax.iota(jnp.int32, 16)

    def abody(i, em):
        return jnp.minimum(em, mbuf[pl.ds(i * 16, 16)])

    em = lax.fori_loop(1, MV, abody, mbuf[pl.ds(0, 16)], unroll=8)
    tvec = jnp.full((16,), jnp.max(em) + MARGIN, jnp.float32)

    def bbody(mb, ptr):
        mv = mbuf[pl.ds(mb * 16, 16)]
        msk = mv <= tvec
        bidv = mb * 16 + lane
        plsc.store_compressed(bids.at[pl.ds(ptr, 16)], bidv, mask=msk)
        cnt = plsc.all_reduce_population_count(msk)
        return ptr + cnt[0]

    nh = lax.fori_loop(0, MV, bbody, 0, unroll=4)

    def merge_into(kv, ki, nk_raw, ni_raw):
        nk, ni = plsc.sort_key_val(nk_raw, ni_raw, descending=True)
        lt = _lex_lt(nk, ni, kv, ki)
        lk = jnp.where(lt, nk, kv)
        li = jnp.where(lt, ni, ki)
        kv2, ki2 = plsc.sort_key_val(lk, li)
        return kv2, ki2

    kv0 = jnp.full((16,), jnp.inf, jnp.float32)
    ki0 = jnp.zeros((16,), jnp.int32)

    def cbody(hi, ptr):
        bid = bids[pl.ds(hi, 16)][0]
        v = dbuf[pl.ds(bid * 16, 16)]
        msk = v <= tvec
        p = jnp.minimum(ptr, CCAP)
        plsc.store_compressed(cvals.at[pl.ds(p, 16)], v, mask=msk)
        plsc.store_compressed(cidx.at[pl.ds(p, 16)], bid * 16 + lane,
                              mask=msk)
        cnt = plsc.all_reduce_population_count(msk)
        return ptr + cnt[0]

    nc = lax.fori_loop(0, nh, cbody, 0)

    def fast(kv, ki):
        cvals[pl.ds(nc, 16)] = kv0
        cidx[pl.ds(nc, 16)] = ki0

        def fbody(ci, carry):
            kv, ki = carry
            return merge_into(kv, ki, cvals[pl.ds(ci * 16, 16)],
                              cidx[pl.ds(ci * 16, 16)])

        return lax.fori_loop(0, (nc + 15) // 16, fbody, (kv, ki))

    def slow(kv, ki):
        def sbody(hi, carry):
            kv, ki = carry
            bid = bids[pl.ds(hi, 16)][0]
            v = dbuf[pl.ds(bid * 16, 16)]
            return merge_into(kv, ki, v, bid * 16 + lane)

        return lax.fori_loop(0, nh, sbody, (kv, ki))

    kv, ki = lax.cond(nc <= CCAP, fast, slow, kv0, ki0)
    return kv, ki


def _topk_sc(d, m):
    mesh = plsc.VectorSubcoreMesh(core_axis_name="c", subcore_axis_name="s")

    @functools.partial(
        pl.kernel,
        mesh=mesh,
        compiler_params=pltpu.CompilerParams(needs_layout_passes=False),
        out_type=jax.ShapeDtypeStruct((N * K,), jnp.float32),
        scratch_types=[
            pltpu.VMEM((NPAD,), jnp.float32),
            pltpu.VMEM((NPAD,), jnp.float32),
            pltpu.VMEM((NB,), jnp.float32),
            pltpu.VMEM((NB,), jnp.float32),
            pltpu.VMEM((NB + 16,), jnp.int32),
            pltpu.VMEM((CCAP + 32,), jnp.float32),
            pltpu.VMEM((CCAP + 32,), jnp.int32),
            pltpu.VMEM((RPW * K,), jnp.float32),
            pltpu.SemaphoreType.DMA,
            pltpu.SemaphoreType.DMA,
            pltpu.SemaphoreType.DMA,
            pltpu.SemaphoreType.DMA,
        ],
    )
    def k(d_hbm, m_hbm, out_hbm, dbuf0, dbuf1, mbuf0, mbuf1, bids, cvals,
          cidx, out_v, dsem0, dsem1, msem0, msem1):
        wid = lax.axis_index("s") * 2 + lax.axis_index("c")
        row0 = wid * RPW
        dbufs = (dbuf0, dbuf1)
        msems = (msem0, msem1)
        mbufs = (mbuf0, mbuf1)
        dsems = (dsem0, dsem1)

        pltpu.async_copy(d_hbm.at[row0], dbuf0, dsem0)
        pltpu.async_copy(m_hbm.at[row0], mbuf0, msem0)

        def outer(i2, _):
            for b in range(2):
                r = i2 * 2 + b
                row = row0 + r
                pltpu.async_copy(d_hbm.at[row + 1], dbufs[1 - b], dsems[1 - b])
                pltpu.async_copy(m_hbm.at[row + 1], mbufs[1 - b], msems[1 - b])
                pltpu.make_async_copy(d_hbm.at[row], dbufs[b], dsems[b]).wait()
                pltpu.make_async_copy(m_hbm.at[row], mbufs[b], msems[b]).wait()
                kv, ki = _row_topk(dbufs[b], mbufs[b], bids, cvals, cidx)

                @pl.when(r < RPW)
                def _():
                    out_v[pl.ds(r * K, K)] = ki.astype(jnp.float32)

            return 0

        lax.fori_loop(0, (RPW + 2) // 2, outer, 0)
        pltpu.make_async_copy(d_hbm.at[row0 + RPW + 1], dbuf0, dsem0).wait()
        pltpu.make_async_copy(m_hbm.at[row0 + RPW + 1], mbuf0, msem0).wait()
        pltpu.sync_copy(out_v, out_hbm.at[pl.ds(row0 * K, RPW * K)])

    out = k(d, m)
    return out.reshape(N, K)


def kernel(barycenters):
    a_pad = jnp.zeros((NPAD, 3), jnp.float32)
    a_pad = a_pad.at[:N].set(barycenters)
    bt_pad = jnp.zeros((3, NPAD), jnp.float32)
    bt_pad = bt_pad.at[:, :N].set(barycenters.T)
    d = _distances(a_pad, bt_pad)
    m = _blockmins(a_pad, bt_pad)
    return _topk_sc(d, m)


# speedup vs baseline: 10.1042x; 1.0288x over previous
import functools

import jax
import jax.numpy as jnp
from jax import lax
from jax.experimental import pallas as pl
from jax.experimental.pallas import tpu as pltpu
from jax.experimental.pallas import tpu_sc as plsc

N = 20000
NPAD = 20480
K = 16
BR = 512
BC = 2048
GW = 32
NB = NPAD // GW
MARGIN = 1e-4

CCAP = 1024
NW = 32
RPW = N // NW
MV = NB // 16


def _sq_dist(a_blk, bt_blk):
    ab = lax.dot_general(
        a_blk, bt_blk,
        dimension_numbers=(((1,), (0,)), ((), ())),
        preferred_element_type=jnp.float32,
    )
    a2 = jnp.sum(a_blk * a_blk, axis=1, keepdims=True)
    b2 = jnp.sum(bt_blk * bt_blk, axis=0)[None, :]
    return (a2 + b2) - 2.0 * ab


def _dist_body(a_ref, bt_ref, o_ref):
    j = pl.program_id(1)
    d = jnp.sqrt(jnp.maximum(_sq_dist(a_ref[...], bt_ref[...]), 0.0))
    col = j * BC + lax.broadcasted_iota(jnp.int32, (BR, BC), 1)
    o_ref[...] = jnp.where(col >= N, jnp.inf, d)


def _distances(a_pad, bt_pad):
    return pl.pallas_call(
        _dist_body,
        grid=(NPAD // BR, NPAD // BC),
        in_specs=[
            pl.BlockSpec((BR, 3), lambda i, j: (i, 0)),
            pl.BlockSpec((3, BC), lambda i, j: (0, j)),
        ],
        out_specs=pl.BlockSpec((BR, BC), lambda i, j: (i, j)),
        out_shape=jax.ShapeDtypeStruct((NPAD, NPAD), jnp.float32),
    )(a_pad, bt_pad)


BCM = 4096


def _mblock_body(a_ref, bt_ref, m_ref):
    jc = pl.program_id(1)
    sq = _sq_dist(a_ref[...], bt_ref[...])
    cc = jc * BCM + lax.broadcasted_iota(jnp.int32, (BCM, BR), 0)
    sq = jnp.where(cc >= N, jnp.inf, sq)
    gm = jnp.min(sq.reshape(BCM // GW, GW, BR), axis=1)
    m_ref[...] = jnp.sqrt(jnp.maximum(gm, 0.0)).T


def _blockmins(a_pad, bt_pad):
    return pl.pallas_call(
        _mblock_body,
        grid=(NPAD // BR, NPAD // BCM),
        in_specs=[
            pl.BlockSpec((BCM, 3), lambda i, j: (j, 0)),
            pl.BlockSpec((3, BR), lambda i, j: (0, i)),
        ],
        out_specs=pl.BlockSpec((BR, BCM // GW), lambda i, j: (i, j)),
        out_shape=jax.ShapeDtypeStruct((NPAD, NB), jnp.float32),
    )(a_pad, bt_pad)


def _lex_lt(ka, ia, kb, ib):
    return (ka < kb) | ((ka == kb) & (ia < ib))


def _row_topk(dbuf, mbuf, bids, cvals, cidx):
    lane = lax.iota(jnp.int32, 16)

    def abody(i, em):
        return jnp.minimum(em, mbuf[pl.ds(i * 16, 16)])

    em = lax.fori_loop(1, MV, abody, mbuf[pl.ds(0, 16)], unroll=8)
    tvec = jnp.full((16,), jnp.max(em) + MARGIN, jnp.float32)

    def bbody(mb, ptr):
        mv = mbuf[pl.ds(mb * 16, 16)]
        msk = mv <= tvec
        bidv = mb * 16 + lane
        plsc.store_compressed(bids.at[pl.ds(ptr, 16)], bidv, mask=msk)
        cnt = plsc.all_reduce_population_count(msk)
        return ptr + cnt[0]

    nh = lax.fori_loop(0, MV, bbody, 0, unroll=4)

    def merge_into(kv, ki, nk_raw, ni_raw):
        nk, ni = plsc.sort_key_val(nk_raw, ni_raw, descending=True)
        lt = _lex_lt(nk, ni, kv, ki)
        lk = jnp.where(lt, nk, kv)
        li = jnp.where(lt, ni, ki)
        kv2, ki2 = plsc.sort_key_val(lk, li)
        return kv2, ki2

    kv0 = jnp.full((16,), jnp.inf, jnp.float32)
    ki0 = jnp.zeros((16,), jnp.int32)

    def cbody(hi, ptr):
        bid = bids[pl.ds(hi, 16)][0]
        for h in range(GW // 16):
            v = dbuf[pl.ds(bid * GW + h * 16, 16)]
            msk = v <= tvec
            p = jnp.minimum(ptr, CCAP)
            plsc.store_compressed(cvals.at[pl.ds(p, 16)], v, mask=msk)
            plsc.store_compressed(cidx.at[pl.ds(p, 16)],
                                  bid * GW + h * 16 + lane, mask=msk)
            cnt = plsc.all_reduce_population_count(msk)
            ptr = ptr + cnt[0]
        return ptr

    nc = lax.fori_loop(0, nh, cbody, 0)

    def fast(kv, ki):
        cvals[pl.ds(nc, 16)] = kv0
        cidx[pl.ds(nc, 16)] = ki0

        def fbody(ci, carry):
            kv, ki = carry
            return merge_into(kv, ki, cvals[pl.ds(ci * 16, 16)],
                              cidx[pl.ds(ci * 16, 16)])

        return lax.fori_loop(0, (nc + 15) // 16, fbody, (kv, ki))

    def slow(kv, ki):
        def sbody(hi, carry):
            kv, ki = carry
            bid = bids[pl.ds(hi, 16)][0]
            for h in range(GW // 16):
                v = dbuf[pl.ds(bid * GW + h * 16, 16)]
                kv, ki = merge_into(kv, ki, v, bid * GW + h * 16 + lane)
            return kv, ki

        return lax.fori_loop(0, nh, sbody, (kv, ki))

    kv, ki = lax.cond(nc <= CCAP, fast, slow, kv0, ki0)
    return kv, ki


def _topk_sc(d, m):
    mesh = plsc.VectorSubcoreMesh(core_axis_name="c", subcore_axis_name="s")

    @functools.partial(
        pl.kernel,
        mesh=mesh,
        compiler_params=pltpu.CompilerParams(needs_layout_passes=False),
        out_type=jax.ShapeDtypeStruct((N * K,), jnp.float32),
        scratch_types=[
            pltpu.VMEM((NPAD,), jnp.float32),
            pltpu.VMEM((NPAD,), jnp.float32),
            pltpu.VMEM((NB,), jnp.float32),
            pltpu.VMEM((NB,), jnp.float32),
            pltpu.VMEM((NB + 16,), jnp.int32),
            pltpu.VMEM((CCAP + 32,), jnp.float32),
            pltpu.VMEM((CCAP + 32,), jnp.int32),
            pltpu.VMEM((RPW * K,), jnp.float32),
            pltpu.SemaphoreType.DMA,
            pltpu.SemaphoreType.DMA,
            pltpu.SemaphoreType.DMA,
            pltpu.SemaphoreType.DMA,
        ],
    )
    def k(d_hbm, m_hbm, out_hbm, dbuf0, dbuf1, mbuf0, mbuf1, bids, cvals,
          cidx, out_v, dsem0, dsem1, msem0, msem1):
        wid = lax.axis_index("s") * 2 + lax.axis_index("c")
        row0 = wid * RPW
        dbufs = (dbuf0, dbuf1)
        msems = (msem0, msem1)
        mbufs = (mbuf0, mbuf1)
        dsems = (dsem0, dsem1)

        pltpu.async_copy(d_hbm.at[row0], dbuf0, dsem0)
        pltpu.async_copy(m_hbm.at[row0], mbuf0, msem0)

        def outer(i2, _):
            for b in range(2):
                r = i2 * 2 + b
                row = row0 + r
                pltpu.async_copy(d_hbm.at[row + 1], dbufs[1 - b], dsems[1 - b])
                pltpu.async_copy(m_hbm.at[row + 1], mbufs[1 - b], msems[1 - b])
                pltpu.make_async_copy(d_hbm.at[row], dbufs[b], dsems[b]).wait()
                pltpu.make_async_copy(m_hbm.at[row], mbufs[b], msems[b]).wait()
                kv, ki = _row_topk(dbufs[b], mbufs[b], bids, cvals, cidx)

                @pl.when(r < RPW)
                def _():
                    out_v[pl.ds(r * K, K)] = ki.astype(jnp.float32)

            return 0

        lax.fori_loop(0, (RPW + 2) // 2, outer, 0)
        pltpu.make_async_copy(d_hbm.at[row0 + RPW + 1], dbuf0, dsem0).wait()
        pltpu.make_async_copy(m_hbm.at[row0 + RPW + 1], mbuf0, msem0).wait()
        pltpu.sync_copy(out_v, out_hbm.at[pl.ds(row0 * K, RPW * K)])

    out = k(d, m)
    return out.reshape(N, K)


def kernel(barycenters):
    a_pad = jnp.zeros((NPAD, 3), jnp.float32)
    a_pad = a_pad.at[:N].set(barycenters)
    bt_pad = jnp.zeros((3, NPAD), jnp.float32)
    bt_pad = bt_pad.at[:, :N].set(barycenters.T)
    d = _distances(a_pad, bt_pad)
    m = _blockmins(a_pad, bt_pad)
    return _topk_sc(d, m)
